# speedup vs baseline: 4.4854x; 4.4854x over previous
import math

import jax
import jax.numpy as jnp
from jax import lax
from jax.experimental import pallas as pl
from jax.experimental.pallas import tpu as pltpu
from jax.experimental.pallas import tpu_sc as plsc

D_MODEL = 128
SCALE = math.sqrt(D_MODEL)
NUM_WORKERS = 32
CHUNK = 128
LANES = 16


def _sc_body(x_hbm, table_hbm, out_hbm, idx_v, rows_v, gsem):
    b_per_w = x_hbm.shape[0] // NUM_WORKERS
    steps = b_per_w // CHUNK
    wid = lax.axis_index("s") * 2 + lax.axis_index("c")
    base = wid * b_per_w

    @pl.loop(0, steps)
    def _step(i):
        row0 = base + i * CHUNK
        pltpu.sync_copy(x_hbm.at[pl.ds(row0, CHUNK)], idx_v)
        pltpu.async_copy(table_hbm.at[idx_v], rows_v, gsem).wait()

        @pl.loop(0, CHUNK)
        def _scale(r):
            for c in range(D_MODEL // LANES):
                s = pl.ds(c * LANES, LANES)
                rows_v[r, s] = rows_v[r, s] * SCALE

        pltpu.sync_copy(rows_v, out_hbm.at[pl.ds(row0, CHUNK)])


def kernel(x, table):
    B = x.shape[0] * x.shape[1]
    xf = x.reshape(B).astype(jnp.int32)
    mesh = plsc.VectorSubcoreMesh(core_axis_name="c", subcore_axis_name="s")
    k = pl.kernel(
        _sc_body,
        out_type=jax.ShapeDtypeStruct((B, D_MODEL), jnp.float32),
        mesh=mesh,
        scratch_types=[
            pltpu.VMEM((CHUNK,), jnp.int32),
            pltpu.VMEM((CHUNK, D_MODEL), jnp.float32),
            pltpu.SemaphoreType.DMA,
        ],
    )
    out = k(xf, table)
    return out.reshape(x.shape + (D_MODEL,))


# speedup vs baseline: 6.4247x; 1.4323x over previous
import math

import jax
import jax.numpy as jnp
from jax import lax
from jax.experimental import pallas as pl
from jax.experimental.pallas import tpu as pltpu
from jax.experimental.pallas import tpu_sc as plsc

D_MODEL = 128
SCALE = math.sqrt(D_MODEL)
NUM_WORKERS = 32
CHUNK = 128
LANES = 16


def _sc_body(x_hbm, table_hbm, out_hbm, idx_v, rows_v, gsems, ssems):
    b_per_w = x_hbm.shape[0] // NUM_WORKERS
    steps = b_per_w // CHUNK
    wid = lax.axis_index("s") * 2 + lax.axis_index("c")
    base = wid * b_per_w
    gsem0, gsem1 = gsems
    ssem0, ssem1 = ssems

    def start_gather(i, slot, gsem):
        pltpu.sync_copy(x_hbm.at[pl.ds(base + i * CHUNK, CHUNK)],
                        idx_v.at[slot])
        pltpu.async_copy(table_hbm.at[idx_v.at[slot]], rows_v.at[slot], gsem)

    def scale(slot):
        @pl.loop(0, CHUNK)
        def _scale(r):
            for c in range(D_MODEL // LANES):
                s = pl.ds(c * LANES, LANES)
                rows_v[slot, r, s] = rows_v[slot, r, s] * SCALE

    def start_scatter(i, slot, ssem):
        pltpu.async_copy(rows_v.at[slot],
                         out_hbm.at[pl.ds(base + i * CHUNK, CHUNK)], ssem)

    def wait_gather(slot, gsem):
        pltpu.make_async_copy(table_hbm.at[idx_v.at[slot]], rows_v.at[slot],
                              gsem).wait()

    def wait_scatter(slot, ssem):
        pltpu.make_async_copy(rows_v.at[slot],
                              out_hbm.at[pl.ds(base, CHUNK)], ssem).wait()

    start_gather(0, 0, gsem0)
    wait_gather(0, gsem0)
    start_gather(1, 1, gsem1)
    scale(0)
    start_scatter(0, 0, ssem0)

    @pl.loop(1, steps - 1, step=2)
    def _step(g):
        for b in range(2):
            i = g + b
            s = (1 + b) % 2
            gsem = gsem1 if s else gsem0
            gsem_n = gsem0 if s else gsem1
            ssem = ssem1 if s else ssem0
            ssem_n = ssem0 if s else ssem1
            wait_gather(s, gsem)
            wait_scatter(1 - s, ssem_n)
            start_gather(i + 1, 1 - s, gsem_n)
            scale(s)
            start_scatter(i, s, ssem)

    wait_gather(1, gsem1)
    wait_scatter(0, ssem0)
    scale(1)
    pltpu.sync_copy(rows_v.at[1],
                    out_hbm.at[pl.ds(base + (steps - 1) * CHUNK, CHUNK)])


def kernel(x, table):
    B = x.shape[0] * x.shape[1]
    xf = x.reshape(B).astype(jnp.int32)
    mesh = plsc.VectorSubcoreMesh(core_axis_name="c", subcore_axis_name="s")
    k = pl.kernel(
        _sc_body,
        out_type=jax.ShapeDtypeStruct((B, D_MODEL), jnp.float32),
        mesh=mesh,
        scratch_types=[
            pltpu.VMEM((2, CHUNK), jnp.int32),
            pltpu.VMEM((2, CHUNK, D_MODEL), jnp.float32),
            [pltpu.SemaphoreType.DMA, pltpu.SemaphoreType.DMA],
            [pltpu.SemaphoreType.DMA, pltpu.SemaphoreType.DMA],
        ],
    )
    out = k(xf, table)
    return out.reshape(x.shape + (D_MODEL,))
